# baseline (speedup 1.0000x reference)
import functools

import jax
import jax.numpy as jnp
from jax import lax
from jax.experimental import pallas as pl
from jax.experimental.pallas import tpu as pltpu
from jax.experimental.pallas import tpu_sc as plsc

EMB_DIM = 64
N_NEGS = 16384
NUM_CORES = 2
NUM_SUBCORES = 16
NUM_WORKERS = NUM_CORES * NUM_SUBCORES
B_PER_W = N_NEGS // NUM_WORKERS
CHUNK = 128
N_CHUNKS = B_PER_W // CHUNK


def _gather_body(emb_hbm, negix_hbm, uvix_hbm, negs_out, uv_out,
                 idx_v, rows_v, uvidx_v, uvrows_v, sem):
    c = lax.axis_index("c")
    s = lax.axis_index("s")
    wid = s * NUM_CORES + c
    base = wid * B_PER_W
    pltpu.sync_copy(negix_hbm.at[pl.ds(base, B_PER_W)], idx_v)
    copies = [
        pltpu.async_copy(
            emb_hbm.at[idx_v.at[pl.ds(j * CHUNK, CHUNK)]],
            rows_v.at[pl.ds(j * CHUNK, CHUNK)],
            sem,
        )
        for j in range(N_CHUNKS)
    ]
    for cp in copies:
        cp.wait()
    pltpu.sync_copy(rows_v, negs_out.at[pl.ds(base, B_PER_W)])

    @pl.when(wid == 0)
    def _():
        pltpu.sync_copy(uvix_hbm, uvidx_v)
        pltpu.async_copy(emb_hbm.at[uvidx_v], uvrows_v, sem).wait()
        pltpu.sync_copy(uvrows_v, uv_out)


_gather = functools.partial(
    pl.kernel,
    out_type=(
        jax.ShapeDtypeStruct((N_NEGS, EMB_DIM), jnp.float32),
        jax.ShapeDtypeStruct((8, EMB_DIM), jnp.float32),
    ),
    mesh=plsc.VectorSubcoreMesh(core_axis_name="c", subcore_axis_name="s"),
    scratch_types=(
        pltpu.VMEM((B_PER_W,), jnp.int32),
        pltpu.VMEM((B_PER_W, EMB_DIM), jnp.float32),
        pltpu.VMEM((8,), jnp.int32),
        pltpu.VMEM((8, EMB_DIM), jnp.float32),
        pltpu.SemaphoreType.DMA,
    ),
    compiler_params=pltpu.CompilerParams(use_tc_tiling_on_sc=False),
)(_gather_body)


def _loss_body(negs_ref, uv_ref, out_ref):
    u = uv_ref[0:1, :]
    v = uv_ref[1:2, :]
    eps = 1e-5
    uu = jnp.sum(u * u)
    vv = jnp.sum(v * v)
    alpha = jnp.clip(1.0 - uu, eps, 1.0)
    beta_v = jnp.clip(1.0 - vv, eps, 1.0)
    sq_uv = jnp.sum((u - v) ** 2)
    gamma_uv = jnp.clip(1.0 + 2.0 * sq_uv / (alpha * beta_v), 1.0 + 1e-7, None)
    d_uv = jnp.log(gamma_uv + jnp.sqrt(gamma_uv * gamma_uv - 1.0))

    negs = negs_ref[...]
    nn = jnp.sum(negs * negs, axis=1, keepdims=True)
    beta_n = jnp.clip(1.0 - nn, eps, 1.0)
    sq_n = jnp.sum((negs - u) ** 2, axis=1, keepdims=True)
    gamma_n = jnp.clip(1.0 + 2.0 * sq_n / (alpha * beta_n), 1.0 + 1e-7, None)
    e_n = gamma_n - jnp.sqrt(gamma_n * gamma_n - 1.0)
    s_sum = jnp.sum(e_n)
    out_ref[...] = jnp.broadcast_to(d_uv + jnp.log(s_sum), (1, 1))


_loss = pl.pallas_call(
    _loss_body,
    out_shape=jax.ShapeDtypeStruct((1, 1), jnp.float32),
)


def kernel(embeddings, u_ix, v_ix, neg_ixs):
    neg_ixs = neg_ixs.astype(jnp.int32)
    u_ix = jnp.asarray(u_ix, jnp.int32)
    v_ix = jnp.asarray(v_ix, jnp.int32)
    uvix = jnp.stack([u_ix, v_ix, u_ix, u_ix, u_ix, u_ix, u_ix, u_ix])
    negs, uv = _gather(embeddings, neg_ixs, uvix)
    loss = _loss(negs, uv)
    u = uv[0:1, :]
    v = uv[1:2, :]
    return (loss, u, v, negs)
